# Initial kernel scaffold; baseline (speedup 1.0000x reference)
import jax
import jax.numpy as jnp
from jax.experimental import pallas as pl


def kernel(e, cross_edge_index, tau):
    raise NotImplementedError("write your pallas kernel here")


# speedup vs baseline: 81.4655x; 81.4655x over previous
import functools
import jax
import jax.numpy as jnp
from jax import lax
from jax.experimental import pallas as pl
from jax.experimental.pallas import tpu as pltpu
from jax.experimental.pallas import tpu_sc as plsc

_E = 6400000
_NV = 100000
_NVP = 100352
_MSL = _NVP // 32
_NW = 32
_EPW = _E // _NW
_CH = 4000
_NCH = _EPW // _CH
_NGR = _CH // 16

_R = 50000
_C = 128
_BLK = 2000
_NB = _R // _BLK

def _gumbel_noise():
    u = jax.random.uniform(
        jax.random.fold_in(jax.random.key(42), 1), (_E,), dtype=jnp.float32,
        minval=1e-20, maxval=1.0)
    return -jnp.log(-jnp.log(u)).reshape(_R, _C)


def _zstat_body(tau_ref, e_ref, g_ref, z_ref, m_ref, s_ref):
    z = (e_ref[...] + g_ref[...]) / tau_ref[0, 0]
    z_ref[...] = z
    m = jnp.max(z)
    m_ref[...] = jnp.full((8, _C), m, jnp.float32)
    s_ref[...] = jnp.full((8, _C), jnp.sum(jnp.exp(z - m)), jnp.float32)


def _norm_body(m_ref, s_ref, z_ref, y_ref):
    y_ref[...] = jnp.exp(z_ref[...] - m_ref[0, 0]) / s_ref[0, 0]


def _scalar_spec():
    return pl.BlockSpec((1, 1), lambda i: (0, 0), memory_space=pltpu.SMEM)


def _blk_spec():
    return pl.BlockSpec((_BLK, _C), lambda i: (i, 0))


def _segmax_body(idx_hbm, z_hbm, out_hbm, table, idxb, zb):
    cid = lax.axis_index("c")
    sid = lax.axis_index("s")
    wid = sid * 2 + cid
    base = wid * _EPW

    neg = jnp.full((16,), -jnp.inf, jnp.float32)

    def _init(i, carry):
        table[pl.ds(i * 16, 16)] = neg
        return carry

    lax.fori_loop(0, _NVP // 16, _init, 0)

    def _chunk(c, carry):
        off = base + c * _CH
        pltpu.sync_copy(idx_hbm.at[pl.ds(off, _CH)], idxb)
        pltpu.sync_copy(z_hbm.at[pl.ds(off, _CH)], zb)

        def _group(g, carry2):
            k = idxb[pl.ds(g * 16, 16)]
            v = zb[pl.ds(g * 16, 16)]
            vs, ks = plsc.sort_key_val(v, k)
            _, last = plsc.scan_count(ks)
            t = plsc.load_gather(table, [ks])
            nv = jnp.maximum(t, vs)
            plsc.store_scatter(table, [ks], nv, mask=last)
            return carry2

        lax.fori_loop(0, _NGR, _group, 0, unroll=8)
        return carry

    lax.fori_loop(0, _NCH, _chunk, 0)

    pltpu.sync_copy(table, out_hbm.at[pl.ds(wid * _NVP, _NVP)])


def _mergemax_body(tabs_hbm, out_hbm, acc, tmpb):
    cid = lax.axis_index("c")
    sid = lax.axis_index("s")
    wid = sid * 2 + cid
    soff = wid * _MSL
    pltpu.sync_copy(tabs_hbm.at[pl.ds(soff, _MSL)], acc)

    def _mtab(t, carry):
        pltpu.sync_copy(tabs_hbm.at[pl.ds(t * _NVP + soff, _MSL)], tmpb)

        def _mg(j, carry2):
            sl = pl.ds(j * 16, 16)
            acc[sl] = jnp.maximum(acc[sl], tmpb[sl])
            return carry2

        lax.fori_loop(0, _MSL // 16, _mg, 0, unroll=8)
        return carry

    lax.fori_loop(1, _NW, _mtab, 0)
    pltpu.sync_copy(acc, out_hbm.at[pl.ds(soff, _MSL)])


def _onehot_body(idx_hbm, z_hbm, segmax_hbm, yh_hbm, table, idxb, zb, ob):
    cid = lax.axis_index("c")
    sid = lax.axis_index("s")
    wid = sid * 2 + cid
    base = wid * _EPW
    pltpu.sync_copy(segmax_hbm, table)

    def _chunk(c, carry):
        off = base + c * _CH
        pltpu.sync_copy(idx_hbm.at[pl.ds(off, _CH)], idxb)
        pltpu.sync_copy(z_hbm.at[pl.ds(off, _CH)], zb)

        def _group(g, carry2):
            sl = pl.ds(g * 16, 16)
            k = idxb[sl]
            v = zb[sl]
            t = plsc.load_gather(table, [k])
            ob[sl] = jnp.where(v == t, jnp.float32(1.0), jnp.float32(0.0))
            return carry2

        lax.fori_loop(0, _NGR, _group, 0, unroll=8)
        pltpu.sync_copy(ob, yh_hbm.at[pl.ds(off, _CH)])
        return carry

    lax.fori_loop(0, _NCH, _chunk, 0)


_SC_CACHE = []


def _sc_kernels():
    if not _SC_CACHE:
        mesh = plsc.VectorSubcoreMesh(core_axis_name="c", subcore_axis_name="s")
        params = pltpu.CompilerParams(needs_layout_passes=False)
        segmax = pl.kernel(
            _segmax_body,
            out_type=jax.ShapeDtypeStruct((_NW * _NVP,), jnp.float32),
            mesh=mesh,
            compiler_params=params,
            scratch_types=[
                pltpu.VMEM((_NVP,), jnp.float32),
                pltpu.VMEM((_CH,), jnp.int32),
                pltpu.VMEM((_CH,), jnp.float32),
            ],
        )
        mergemax = pl.kernel(
            _mergemax_body,
            out_type=jax.ShapeDtypeStruct((_NVP,), jnp.float32),
            mesh=mesh,
            compiler_params=params,
            scratch_types=[
                pltpu.VMEM((_MSL,), jnp.float32),
                pltpu.VMEM((_MSL,), jnp.float32),
            ],
        )
        onehot = pl.kernel(
            _onehot_body,
            out_type=jax.ShapeDtypeStruct((_E,), jnp.float32),
            mesh=mesh,
            compiler_params=params,
            scratch_types=[
                pltpu.VMEM((_NVP,), jnp.float32),
                pltpu.VMEM((_CH,), jnp.int32),
                pltpu.VMEM((_CH,), jnp.float32),
                pltpu.VMEM((_CH,), jnp.float32),
            ],
        )
        _SC_CACHE.append((segmax, mergemax, onehot))
    return _SC_CACHE[0]


def kernel(e, cross_edge_index, tau):
    tau2 = jnp.asarray(tau, jnp.float32).reshape(1, 1)
    e2 = e.reshape(_R, _C)

    z2, pm, ps = pl.pallas_call(
        _zstat_body,
        grid=(_NB,),
        in_specs=[_scalar_spec(), _blk_spec(), _blk_spec()],
        out_specs=[
            _blk_spec(),
            pl.BlockSpec((8, _C), lambda i: (i, 0)),
            pl.BlockSpec((8, _C), lambda i: (i, 0)),
        ],
        out_shape=[
            jax.ShapeDtypeStruct((_R, _C), jnp.float32),
            jax.ShapeDtypeStruct((_NB * 8, _C), jnp.float32),
            jax.ShapeDtypeStruct((_NB * 8, _C), jnp.float32),
        ],
    )(tau2, e2, _gumbel_noise())

    pmv = pm[::8, 0]
    psv = ps[::8, 0]
    m = jnp.max(pmv)
    s = jnp.sum(psv * jnp.exp(pmv - m))

    y2 = pl.pallas_call(
        _norm_body,
        grid=(_NB,),
        in_specs=[_scalar_spec(), _scalar_spec(), _blk_spec()],
        out_specs=_blk_spec(),
        out_shape=jax.ShapeDtypeStruct((_R, _C), jnp.float32),
    )(m.reshape(1, 1), s.reshape(1, 1), z2)

    z1 = z2.reshape(_E)
    idx = cross_edge_index[1]
    segmax_k, mergemax_k, onehot_k = _sc_kernels()
    tabs = segmax_k(idx, z1)
    segmax = mergemax_k(tabs)
    yh = onehot_k(idx, z1, segmax)
    return (y2.reshape(_E)[:, None], yh[:, None])
